# baseline (speedup 1.0000x reference)
import functools

import jax
import jax.numpy as jnp
from jax import lax
from jax.experimental import pallas as pl
from jax.experimental.pallas import tpu as pltpu
from jax.experimental.pallas import tpu_sc as plsc

_N = 10000
_E = 320000
_D = 128
_U = 128

_NC = 2
_NS = 16
_L = 16
_NW = _NC * _NS
_EPW = _E // _NW
_SC = 64
_EPWP = 10240
_SPS = 20
_NSEG = _EPWP // (_SC * _SPS)
_ZC = 80
_ZT = 10
_RPZ = _N // _ZT


def _lane_bcast(vec, j):
  idx = jnp.full((_L, 1), j, dtype=jnp.int32)
  dnums = lax.GatherDimensionNumbers(
      offset_dims=(), collapsed_slice_dims=(0,), start_index_map=(0,))
  return lax.gather(vec, idx, dnums, (1,),
                    mode=lax.GatherScatterMode.PROMISE_IN_BOUNDS)


def _sc_body(nf, src4, dst4, ew4, aggp, wsump,
             src_v, dst_v, w_v, zb_v, wsb_v, rows0, rows1, rows2,
             agg_sh, ws_sh, g0, g1, g2, s0, s1, s2):
  cid = lax.axis_index("c")
  sid = lax.axis_index("s")
  wid = sid * _NC + cid
  zero = jnp.zeros((_L,), jnp.float32)

  def _zfill(i, carry):
    for k in range(_D // _L):
      rows0[i, pl.ds(k * _L, _L)] = zero
    return carry
  lax.fori_loop(0, _SC, _zfill, 0)
  for k in range(_ZC // _L):
    zb_v[pl.ds(k * _L, _L)] = zero

  @pl.when(sid < _ZT)
  def _zero_acc():
    row0 = sid * _RPZ
    def _zacc(i, carry):
      pltpu.sync_copy(rows0, agg_sh.at[pl.ds(row0 + i * _SC, _SC)])
      return carry
    lax.fori_loop(0, _RPZ // _SC, _zacc, 0)
    rem = _RPZ - (_RPZ // _SC) * _SC
    pltpu.sync_copy(rows0.at[pl.ds(0, rem)],
                    agg_sh.at[pl.ds(row0 + _RPZ - rem, rem)])
    for i in range(_RPZ // _ZC):
      pltpu.sync_copy(zb_v, ws_sh.at[pl.ds(row0 + i * _ZC, _ZC)])
    rem2 = _RPZ - (_RPZ // _ZC) * _ZC
    pltpu.sync_copy(zb_v.at[pl.ds(0, rem2)],
                    ws_sh.at[pl.ds(row0 + _RPZ - rem2, rem2)])

  plsc.subcore_barrier()

  bufs = (rows0, rows1, rows2)
  gsems = (g0, g1, g2)
  ssems = (s0, s1, s2)

  def _g(u, j):
    return pltpu.make_async_copy(nf.at[src_v.at[u]], bufs[j], gsems[j])

  def _s_rows(u, j):
    return pltpu.make_async_copy(bufs[j], agg_sh.at[dst_v.at[u]], ssems[j])

  def _s_w(u, j):
    return pltpu.make_async_copy(w_v.at[u], ws_sh.at[dst_v.at[u]], ssems[j])

  def _scatter_issue(u, j):
    pltpu.async_copy(bufs[j], agg_sh.at[dst_v.at[u]], ssems[j], add=True)
    pltpu.async_copy(w_v.at[u], ws_sh.at[dst_v.at[u]], ssems[j], add=True)

  def _scatter_wait(u, j):
    _s_rows(u, j).wait()
    _s_w(u, j).wait()

  def _scale_rows(u, buf, base, jlo):
    w16 = w_v[u, pl.ds(base, _L)]
    for j in range(jlo, _L):
      r = base + j
      wj = _lane_bcast(w16, j)
      for k in range(_D // _L):
        buf[r, pl.ds(k * _L, _L)] = buf[r, pl.ds(k * _L, _L)] * wj

  def _scale(u, buf):
    def _half(h, c):
      _scale_rows(u, buf, h * 2 * _L, 0)
      _scale_rows(u, buf, h * 2 * _L + _L, 0)
      return c
    lax.fori_loop(0, _SC // (2 * _L), _half, 0)

  def _slot(u, j, gather_ahead, wait_scatter):
    if wait_scatter:
      _scatter_wait(u - 2, (j + 1) % 3)
    if gather_ahead:
      _g(u + 1, (j + 1) % 3).start()
    _g(u, j).wait()
    _scale(u, bufs[j])
    _scatter_issue(u, j)

  def _segment(seg, carry):
    pltpu.sync_copy(src4.at[wid, seg], src_v)
    pltpu.sync_copy(dst4.at[wid, seg], dst_v)
    pltpu.sync_copy(ew4.at[wid, seg], w_v)
    _g(0, 0).start()
    _slot(0, 0, True, False)
    _slot(1, 1, True, False)

    def _tri(i, c):
      base = 2 + 3 * i
      _slot(base + 0, 2, True, True)
      _slot(base + 1, 0, True, True)
      _slot(base + 2, 1, True, True)
      return c

    lax.fori_loop(0, (_SPS - 5) // 3, _tri, 0)
    _slot(_SPS - 3, 2, True, True)
    _slot(_SPS - 2, 0, True, True)
    _slot(_SPS - 1, 1, False, True)
    _scatter_wait(_SPS - 2, 0)
    _scatter_wait(_SPS - 1, 1)
    return carry

  lax.fori_loop(0, _NSEG, _segment, 0)

  plsc.subcore_barrier()

  @pl.when(sid < _ZT)
  def _copy_out():
    row0 = sid * _RPZ
    pltpu.sync_copy(agg_sh.at[pl.ds(row0, _RPZ)],
                    aggp.at[cid, pl.ds(row0, _RPZ)])
    pltpu.sync_copy(ws_sh.at[pl.ds(row0, _RPZ)], wsb_v)
    pltpu.sync_copy(wsb_v, wsump.at[pl.ds(cid * _N + row0, _RPZ)])


_sc_agg = functools.partial(
    pl.kernel,
    out_type=(jax.ShapeDtypeStruct((_NC, _N, _D), jnp.float32),
              jax.ShapeDtypeStruct((_NC * _N,), jnp.float32)),
    mesh=plsc.VectorSubcoreMesh(core_axis_name="c", subcore_axis_name="s"),
    scratch_types=[
        pltpu.VMEM((_SPS, _SC), jnp.int32),
        pltpu.VMEM((_SPS, _SC), jnp.int32),
        pltpu.VMEM((_SPS, _SC), jnp.float32),
        pltpu.VMEM((_ZC,), jnp.float32),
        pltpu.VMEM((_RPZ,), jnp.float32),
        pltpu.VMEM((_SC, _D), jnp.float32),
        pltpu.VMEM((_SC, _D), jnp.float32),
        pltpu.VMEM((_SC, _D), jnp.float32),
        pltpu.VMEM_SHARED((_N, _D), jnp.float32),
        pltpu.VMEM_SHARED((_N,), jnp.float32),
    ] + [pltpu.SemaphoreType.DMA] * 6,
)(_sc_body)


def _tc_body(aggp_ref, ws_ref, w_ref, b_ref, out_ref):
  s = aggp_ref[0] + aggp_ref[1]
  ws = ws_ref[0] + ws_ref[1]
  denom = jnp.where(ws > 0.0, ws, 1.0)
  combined = s / denom
  acc = lax.dot_general(combined, w_ref[...], (((1,), (0,)), ((), ())),
                        preferred_element_type=jnp.float32,
                        precision=lax.Precision.HIGHEST)
  out_ref[...] = jnp.maximum(acc + b_ref[...], 0.0)


_BN = 400


def _tc_finish(aggp, wsum3, W, b2):
  return pl.pallas_call(
      _tc_body,
      grid=(_N // _BN,),
      in_specs=[
          pl.BlockSpec((_NC, _BN, _D), lambda i: (0, i, 0)),
          pl.BlockSpec((_NC, _BN, 1), lambda i: (0, i, 0)),
          pl.BlockSpec((_D, _U), lambda i: (0, 0)),
          pl.BlockSpec((1, _U), lambda i: (0, 0)),
      ],
      out_specs=pl.BlockSpec((_BN, _U), lambda i: (i, 0)),
      out_shape=jax.ShapeDtypeStruct((_N, _U), jnp.float32),
  )(aggp, wsum3, W, b2)


@jax.jit
def _impl(node_features, edge_index, edge_weights, W, b):
  ei = edge_index.astype(jnp.int32)
  pad = ((0, 0), (0, _EPWP - _EPW))
  src4 = jnp.pad(ei[1].reshape(_NW, _EPW), pad).reshape(_NW, _NSEG, _SPS, _SC)
  dst4 = jnp.pad(ei[0].reshape(_NW, _EPW), pad).reshape(_NW, _NSEG, _SPS, _SC)
  ew4 = jnp.pad(edge_weights.reshape(_NW, _EPW),
                pad).reshape(_NW, _NSEG, _SPS, _SC)
  aggp, wsump = _sc_agg(node_features, src4, dst4, ew4)
  return _tc_finish(aggp, wsump.reshape(_NC, _N, 1), W, b.reshape(1, _U))


def kernel(node_features, edge_index, edge_weights, W, b):
  return _impl(node_features, edge_index, edge_weights, W, b)


# speedup vs baseline: 1.8106x; 1.8106x over previous
import functools

import jax
import jax.numpy as jnp
from jax import lax
from jax.experimental import pallas as pl
from jax.experimental.pallas import tpu as pltpu
from jax.experimental.pallas import tpu_sc as plsc

_N = 10000
_E = 320000
_D = 128
_U = 128

_NC = 2
_NS = 16
_L = 16
_NW = _NC * _NS
_EPW = _E // _NW
_C = 80
_CPS = 25
_NSEG = _EPW // (_C * _CPS)
_ZT = 10
_RPZ = _N // _ZT


def _lane_bcast(vec, j):
  idx = jnp.full((_L, 1), j, dtype=jnp.int32)
  dnums = lax.GatherDimensionNumbers(
      offset_dims=(), collapsed_slice_dims=(0,), start_index_map=(0,))
  return lax.gather(vec, idx, dnums, (1,),
                    mode=lax.GatherScatterMode.PROMISE_IN_BOUNDS)


def _sc_body(nf, src4, dst4, ew4, aggp, wsump,
             src_v, dst_v, w_v, zb_v, wsb_v, rows0, rows1, agg_sh, ws_sh,
             g0, g1, s0):
  cid = lax.axis_index("c")
  sid = lax.axis_index("s")
  wid = sid * _NC + cid
  zero = jnp.zeros((_L,), jnp.float32)

  def _zfill(i, carry):
    for k in range(_D // _L):
      rows0[i, pl.ds(k * _L, _L)] = zero
    return carry
  lax.fori_loop(0, _C, _zfill, 0)
  for k in range(_C // _L):
    zb_v[pl.ds(k * _L, _L)] = zero

  @pl.when(sid < _ZT)
  def _zero_acc():
    row0 = sid * _RPZ
    for i in range(_RPZ // _C):
      pltpu.sync_copy(rows0, agg_sh.at[pl.ds(row0 + i * _C, _C)])
      pltpu.sync_copy(zb_v, ws_sh.at[pl.ds(row0 + i * _C, _C)])
    rem = _RPZ - (_RPZ // _C) * _C
    pltpu.sync_copy(rows0.at[pl.ds(0, rem)],
                    agg_sh.at[pl.ds(row0 + _RPZ - rem, rem)])
    pltpu.sync_copy(zb_v.at[pl.ds(0, rem)],
                    ws_sh.at[pl.ds(row0 + _RPZ - rem, rem)])

  plsc.subcore_barrier()

  def _gather(t, buf, sem):
    return pltpu.make_async_copy(nf.at[src_v.at[t]], buf, sem)

  def _scale(t, buf):
    for g in range(_C // _L):
      w16 = w_v[t, pl.ds(g * _L, _L)]
      for j in range(_L):
        r = g * _L + j
        wj = _lane_bcast(w16, j)
        for k in range(_D // _L):
          buf[r, pl.ds(k * _L, _L)] = buf[r, pl.ds(k * _L, _L)] * wj

  def _scatter(t, buf):
    pltpu.async_copy(buf, agg_sh.at[dst_v.at[t]], s0, add=True)
    pltpu.async_copy(w_v.at[t], ws_sh.at[dst_v.at[t]], s0, add=True)
    pltpu.make_async_copy(buf, agg_sh.at[dst_v.at[t]], s0).wait()
    pltpu.make_async_copy(w_v.at[t], ws_sh.at[dst_v.at[t]], s0).wait()

  def _segment(seg, carry):
    pltpu.sync_copy(src4.at[wid, seg], src_v)
    pltpu.sync_copy(dst4.at[wid, seg], dst_v)
    pltpu.sync_copy(ew4.at[wid, seg], w_v)
    _gather(0, rows0, g0).start()

    def _pair(i, c):
      t0 = 2 * i
      t1 = t0 + 1
      _gather(t1, rows1, g1).start()
      _gather(t0, rows0, g0).wait()
      _scale(t0, rows0)
      _scatter(t0, rows0)
      _gather(t0 + 2, rows0, g0).start()
      _gather(t1, rows1, g1).wait()
      _scale(t1, rows1)
      _scatter(t1, rows1)
      return c

    lax.fori_loop(0, (_CPS - 1) // 2, _pair, 0)
    _gather(_CPS - 1, rows0, g0).wait()
    _scale(_CPS - 1, rows0)
    _scatter(_CPS - 1, rows0)
    return carry

  lax.fori_loop(0, _NSEG, _segment, 0)

  plsc.subcore_barrier()

  @pl.when(sid < _ZT)
  def _copy_out():
    row0 = sid * _RPZ
    pltpu.sync_copy(agg_sh.at[pl.ds(row0, _RPZ)],
                    aggp.at[cid, pl.ds(row0, _RPZ)])
    pltpu.sync_copy(ws_sh.at[pl.ds(row0, _RPZ)], wsb_v)
    pltpu.sync_copy(wsb_v, wsump.at[pl.ds(cid * _N + row0, _RPZ)])


_sc_agg = functools.partial(
    pl.kernel,
    out_type=(jax.ShapeDtypeStruct((_NC, _N, _D), jnp.float32),
              jax.ShapeDtypeStruct((_NC * _N,), jnp.float32)),
    mesh=plsc.VectorSubcoreMesh(core_axis_name="c", subcore_axis_name="s"),
    scratch_types=[
        pltpu.VMEM((_CPS, _C), jnp.int32),
        pltpu.VMEM((_CPS, _C), jnp.int32),
        pltpu.VMEM((_CPS, _C), jnp.float32),
        pltpu.VMEM((_C,), jnp.float32),
        pltpu.VMEM((_RPZ,), jnp.float32),
        pltpu.VMEM((_C, _D), jnp.float32),
        pltpu.VMEM((_C, _D), jnp.float32),
        pltpu.VMEM_SHARED((_N, _D), jnp.float32),
        pltpu.VMEM_SHARED((_N,), jnp.float32),
        pltpu.SemaphoreType.DMA,
        pltpu.SemaphoreType.DMA,
        pltpu.SemaphoreType.DMA,
    ],
)(_sc_body)


def _tc_body(aggp_ref, ws_ref, w_ref, b_ref, out_ref):
  s = aggp_ref[0] + aggp_ref[1]
  ws = ws_ref[0] + ws_ref[1]
  denom = jnp.where(ws > 0.0, ws, 1.0)
  combined = s / denom
  acc = lax.dot_general(combined, w_ref[...], (((1,), (0,)), ((), ())),
                        preferred_element_type=jnp.float32,
                        precision=lax.Precision.HIGHEST)
  out_ref[...] = jnp.maximum(acc + b_ref[...], 0.0)


_BN = 400


def _tc_finish(aggp, wsum3, W, b2):
  return pl.pallas_call(
      _tc_body,
      grid=(_N // _BN,),
      in_specs=[
          pl.BlockSpec((_NC, _BN, _D), lambda i: (0, i, 0)),
          pl.BlockSpec((_NC, _BN, 1), lambda i: (0, i, 0)),
          pl.BlockSpec((_D, _U), lambda i: (0, 0)),
          pl.BlockSpec((1, _U), lambda i: (0, 0)),
      ],
      out_specs=pl.BlockSpec((_BN, _U), lambda i: (i, 0)),
      out_shape=jax.ShapeDtypeStruct((_N, _U), jnp.float32),
  )(aggp, wsum3, W, b2)


@jax.jit
def _impl(node_features, edge_index, edge_weights, W, b):
  ei = edge_index.astype(jnp.int32)
  src4 = ei[1].reshape(_NW, _NSEG, _CPS, _C)
  dst4 = ei[0].reshape(_NW, _NSEG, _CPS, _C)
  ew4 = edge_weights.reshape(_NW, _NSEG, _CPS, _C)
  aggp, wsump = _sc_agg(node_features, src4, dst4, ew4)
  return _tc_finish(aggp, wsump.reshape(_NC, _N, 1), W, b.reshape(1, _U))


def kernel(node_features, edge_index, edge_weights, W, b):
  return _impl(node_features, edge_index, edge_weights, W, b)


# speedup vs baseline: 1.9593x; 1.0821x over previous
import functools

import jax
import jax.numpy as jnp
from jax import lax
from jax.experimental import pallas as pl
from jax.experimental.pallas import tpu as pltpu
from jax.experimental.pallas import tpu_sc as plsc

_N = 10000
_E = 320000
_D = 128
_U = 128

_NC = 2
_NS = 16
_L = 16
_NW = _NC * _NS
_EPW = _E // _NW
_C = 80
_CPS = 25
_NSEG = _EPW // (_C * _CPS)
_ZT = 10
_RPZ = _N // _ZT


def _lane_bcast(vec, j):
  idx = jnp.full((_L, 1), j, dtype=jnp.int32)
  dnums = lax.GatherDimensionNumbers(
      offset_dims=(), collapsed_slice_dims=(0,), start_index_map=(0,))
  return lax.gather(vec, idx, dnums, (1,),
                    mode=lax.GatherScatterMode.PROMISE_IN_BOUNDS)


def _sc_body(nf, src4, dst4, ew4, aggp, wsump,
             src_v, dst_v, w_v, zb_v, wsb_v, rows0, rows1, agg_sh, ws_sh,
             g0, g1, s0):
  cid = lax.axis_index("c")
  sid = lax.axis_index("s")
  wid = sid * _NC + cid
  zero = jnp.zeros((_L,), jnp.float32)

  def _zfill(i, carry):
    for k in range(_D // _L):
      rows0[i, pl.ds(k * _L, _L)] = zero
    return carry
  lax.fori_loop(0, _C, _zfill, 0)
  for k in range(_C // _L):
    zb_v[pl.ds(k * _L, _L)] = zero

  @pl.when(sid < _ZT)
  def _zero_acc():
    row0 = sid * _RPZ
    for i in range(_RPZ // _C):
      pltpu.sync_copy(rows0, agg_sh.at[pl.ds(row0 + i * _C, _C)])
      pltpu.sync_copy(zb_v, ws_sh.at[pl.ds(row0 + i * _C, _C)])
    rem = _RPZ - (_RPZ // _C) * _C
    pltpu.sync_copy(rows0.at[pl.ds(0, rem)],
                    agg_sh.at[pl.ds(row0 + _RPZ - rem, rem)])
    pltpu.sync_copy(zb_v.at[pl.ds(0, rem)],
                    ws_sh.at[pl.ds(row0 + _RPZ - rem, rem)])

  plsc.subcore_barrier()

  def _gather(t, buf, sem):
    return pltpu.make_async_copy(nf.at[src_v.at[t]], buf, sem)

  def _scale(t, buf):
    for g in range(_C // _L):
      w16 = w_v[t, pl.ds(g * _L, _L)]
      for j in range(_L):
        r = g * _L + j
        wj = _lane_bcast(w16, j)
        for k in range(_D // _L):
          buf[r, pl.ds(k * _L, _L)] = buf[r, pl.ds(k * _L, _L)] * wj

  def _scatter(t, buf):
    pltpu.sync_copy(buf, agg_sh.at[dst_v.at[t]], add=True)
    pltpu.sync_copy(w_v.at[t], ws_sh.at[dst_v.at[t]], add=True)

  def _segment(seg, carry):
    pltpu.sync_copy(src4.at[wid, seg], src_v)
    pltpu.sync_copy(dst4.at[wid, seg], dst_v)
    pltpu.sync_copy(ew4.at[wid, seg], w_v)
    _gather(0, rows0, g0).start()

    def _pair(i, c):
      t0 = 2 * i
      t1 = t0 + 1
      _gather(t1, rows1, g1).start()
      _gather(t0, rows0, g0).wait()
      _scale(t0, rows0)
      _scatter(t0, rows0)
      _gather(t0 + 2, rows0, g0).start()
      _gather(t1, rows1, g1).wait()
      _scale(t1, rows1)
      _scatter(t1, rows1)
      return c

    lax.fori_loop(0, (_CPS - 1) // 2, _pair, 0)
    _gather(_CPS - 1, rows0, g0).wait()
    _scale(_CPS - 1, rows0)
    _scatter(_CPS - 1, rows0)
    return carry

  lax.fori_loop(0, _NSEG, _segment, 0)

  plsc.subcore_barrier()

  @pl.when(sid < _ZT)
  def _copy_out():
    row0 = sid * _RPZ
    pltpu.sync_copy(agg_sh.at[pl.ds(row0, _RPZ)],
                    aggp.at[cid, pl.ds(row0, _RPZ)])
    pltpu.sync_copy(ws_sh.at[pl.ds(row0, _RPZ)], wsb_v)
    pltpu.sync_copy(wsb_v, wsump.at[pl.ds(cid * _N + row0, _RPZ)])


_sc_agg = functools.partial(
    pl.kernel,
    out_type=(jax.ShapeDtypeStruct((_NC, _N, _D), jnp.float32),
              jax.ShapeDtypeStruct((_NC * _N,), jnp.float32)),
    mesh=plsc.VectorSubcoreMesh(core_axis_name="c", subcore_axis_name="s"),
    scratch_types=[
        pltpu.VMEM((_CPS, _C), jnp.int32),
        pltpu.VMEM((_CPS, _C), jnp.int32),
        pltpu.VMEM((_CPS, _C), jnp.float32),
        pltpu.VMEM((_C,), jnp.float32),
        pltpu.VMEM((_RPZ,), jnp.float32),
        pltpu.VMEM((_C, _D), jnp.float32),
        pltpu.VMEM((_C, _D), jnp.float32),
        pltpu.VMEM_SHARED((_N, _D), jnp.float32),
        pltpu.VMEM_SHARED((_N,), jnp.float32),
        pltpu.SemaphoreType.DMA,
        pltpu.SemaphoreType.DMA,
        pltpu.SemaphoreType.DMA,
    ],
)(_sc_body)


def _tc_body(aggp_ref, ws_ref, w_ref, b_ref, out_ref):
  s = aggp_ref[0] + aggp_ref[1]
  ws = ws_ref[0] + ws_ref[1]
  denom = jnp.where(ws > 0.0, ws, 1.0)
  combined = s / denom
  acc = lax.dot_general(combined, w_ref[...], (((1,), (0,)), ((), ())),
                        preferred_element_type=jnp.float32,
                        precision=lax.Precision.HIGHEST)
  out_ref[...] = jnp.maximum(acc + b_ref[...], 0.0)


_BN = 400


def _tc_finish(aggp, wsum3, W, b2):
  return pl.pallas_call(
      _tc_body,
      grid=(_N // _BN,),
      in_specs=[
          pl.BlockSpec((_NC, _BN, _D), lambda i: (0, i, 0)),
          pl.BlockSpec((_NC, _BN, 1), lambda i: (0, i, 0)),
          pl.BlockSpec((_D, _U), lambda i: (0, 0)),
          pl.BlockSpec((1, _U), lambda i: (0, 0)),
      ],
      out_specs=pl.BlockSpec((_BN, _U), lambda i: (i, 0)),
      out_shape=jax.ShapeDtypeStruct((_N, _U), jnp.float32),
  )(aggp, wsum3, W, b2)


@jax.jit
def _impl(node_features, edge_index, edge_weights, W, b):
  ei = edge_index.astype(jnp.int32)
  src4 = ei[1].reshape(_NW, _NSEG, _CPS, _C)
  dst4 = ei[0].reshape(_NW, _NSEG, _CPS, _C)
  ew4 = edge_weights.reshape(_NW, _NSEG, _CPS, _C)
  aggp, wsump = _sc_agg(node_features, src4, dst4, ew4)
  return _tc_finish(aggp, wsump.reshape(_NC, _N, 1), W, b.reshape(1, _U))


def kernel(node_features, edge_index, edge_weights, W, b):
  return _impl(node_features, edge_index, edge_weights, W, b)


# speedup vs baseline: 1.9699x; 1.0054x over previous
import functools

import jax
import jax.numpy as jnp
from jax import lax
from jax.experimental import pallas as pl
from jax.experimental.pallas import tpu as pltpu
from jax.experimental.pallas import tpu_sc as plsc

_N = 10000
_E = 320000
_D = 128
_U = 128

_NC = 2
_NS = 16
_L = 16
_NW = _NC * _NS
_EPW = _E // _NW
_C = 80
_CPS = 25
_NSEG = _EPW // (_C * _CPS)
_ZT = 10
_RPZ = _N // _ZT


def _lane_bcast(vec, j):
  idx = jnp.full((_L, 1), j, dtype=jnp.int32)
  dnums = lax.GatherDimensionNumbers(
      offset_dims=(), collapsed_slice_dims=(0,), start_index_map=(0,))
  return lax.gather(vec, idx, dnums, (1,),
                    mode=lax.GatherScatterMode.PROMISE_IN_BOUNDS)


def _sc_body(nf, src4, dst4, ew4, aggp, wsump,
             src_v, dst_v, w_v, zb_v, wsb_v, rows0, rows1, agg_sh, ws_sh,
             g0, g1, s0):
  cid = lax.axis_index("c")
  sid = lax.axis_index("s")
  wid = sid * _NC + cid
  zero = jnp.zeros((_L,), jnp.float32)

  def _zfill(i, carry):
    for k in range(_D // _L):
      rows0[i, pl.ds(k * _L, _L)] = zero
    return carry
  lax.fori_loop(0, _C, _zfill, 0)
  for k in range(_C // _L):
    zb_v[pl.ds(k * _L, _L)] = zero

  @pl.when(sid < _ZT)
  def _zero_acc():
    row0 = sid * _RPZ
    for i in range(_RPZ // _C):
      pltpu.sync_copy(rows0, agg_sh.at[pl.ds(row0 + i * _C, _C)])
      pltpu.sync_copy(zb_v, ws_sh.at[pl.ds(row0 + i * _C, _C)])
    rem = _RPZ - (_RPZ // _C) * _C
    pltpu.sync_copy(rows0.at[pl.ds(0, rem)],
                    agg_sh.at[pl.ds(row0 + _RPZ - rem, rem)])
    pltpu.sync_copy(zb_v.at[pl.ds(0, rem)],
                    ws_sh.at[pl.ds(row0 + _RPZ - rem, rem)])

  plsc.subcore_barrier()

  def _gather(t, buf, sem):
    return pltpu.make_async_copy(nf.at[src_v.at[t]], buf, sem)

  def _scale(t, buf):
    for g in range(_C // _L):
      w16 = w_v[t, pl.ds(g * _L, _L)]
      for j in range(_L):
        r = g * _L + j
        wj = _lane_bcast(w16, j)
        for k in range(_D // _L):
          buf[r, pl.ds(k * _L, _L)] = buf[r, pl.ds(k * _L, _L)] * wj

  def _scatter(t, buf):
    pltpu.sync_copy(buf, agg_sh.at[dst_v.at[t]], add=True)
    pltpu.sync_copy(w_v.at[t], ws_sh.at[dst_v.at[t]], add=True)

  def _segment(seg, carry):
    pltpu.sync_copy(src4.at[wid, seg], src_v)
    pltpu.sync_copy(dst4.at[wid, seg], dst_v)
    pltpu.sync_copy(ew4.at[wid, seg], w_v)
    _gather(0, rows0, g0).start()

    def _pair(i, c):
      t0 = 2 * i
      t1 = t0 + 1
      _gather(t1, rows1, g1).start()
      _gather(t0, rows0, g0).wait()
      _scale(t0, rows0)
      pltpu.sync_copy(rows0, agg_sh.at[dst_v.at[t0]], add=True)
      _gather(t0 + 2, rows0, g0).start()
      pltpu.sync_copy(w_v.at[t0], ws_sh.at[dst_v.at[t0]], add=True)
      _gather(t1, rows1, g1).wait()
      _scale(t1, rows1)
      _scatter(t1, rows1)
      return c

    lax.fori_loop(0, (_CPS - 1) // 2, _pair, 0)
    _gather(_CPS - 1, rows0, g0).wait()
    _scale(_CPS - 1, rows0)
    _scatter(_CPS - 1, rows0)
    return carry

  lax.fori_loop(0, _NSEG, _segment, 0)

  plsc.subcore_barrier()

  @pl.when(sid < _ZT)
  def _copy_out():
    row0 = sid * _RPZ
    pltpu.sync_copy(agg_sh.at[pl.ds(row0, _RPZ)],
                    aggp.at[cid, pl.ds(row0, _RPZ)])
    pltpu.sync_copy(ws_sh.at[pl.ds(row0, _RPZ)], wsb_v)
    pltpu.sync_copy(wsb_v, wsump.at[pl.ds(cid * _N + row0, _RPZ)])


_sc_agg = functools.partial(
    pl.kernel,
    out_type=(jax.ShapeDtypeStruct((_NC, _N, _D), jnp.float32),
              jax.ShapeDtypeStruct((_NC * _N,), jnp.float32)),
    mesh=plsc.VectorSubcoreMesh(core_axis_name="c", subcore_axis_name="s"),
    scratch_types=[
        pltpu.VMEM((_CPS, _C), jnp.int32),
        pltpu.VMEM((_CPS, _C), jnp.int32),
        pltpu.VMEM((_CPS, _C), jnp.float32),
        pltpu.VMEM((_C,), jnp.float32),
        pltpu.VMEM((_RPZ,), jnp.float32),
        pltpu.VMEM((_C, _D), jnp.float32),
        pltpu.VMEM((_C, _D), jnp.float32),
        pltpu.VMEM_SHARED((_N, _D), jnp.float32),
        pltpu.VMEM_SHARED((_N,), jnp.float32),
        pltpu.SemaphoreType.DMA,
        pltpu.SemaphoreType.DMA,
        pltpu.SemaphoreType.DMA,
    ],
)(_sc_body)


def _tc_body(aggp_ref, ws_ref, w_ref, b_ref, out_ref):
  s = aggp_ref[0] + aggp_ref[1]
  ws = ws_ref[0] + ws_ref[1]
  denom = jnp.where(ws > 0.0, ws, 1.0)
  combined = s / denom
  acc = lax.dot_general(combined, w_ref[...], (((1,), (0,)), ((), ())),
                        preferred_element_type=jnp.float32,
                        precision=lax.Precision.HIGHEST)
  out_ref[...] = jnp.maximum(acc + b_ref[...], 0.0)


_BN = 400


def _tc_finish(aggp, wsum3, W, b2):
  return pl.pallas_call(
      _tc_body,
      grid=(_N // _BN,),
      in_specs=[
          pl.BlockSpec((_NC, _BN, _D), lambda i: (0, i, 0)),
          pl.BlockSpec((_NC, _BN, 1), lambda i: (0, i, 0)),
          pl.BlockSpec((_D, _U), lambda i: (0, 0)),
          pl.BlockSpec((1, _U), lambda i: (0, 0)),
      ],
      out_specs=pl.BlockSpec((_BN, _U), lambda i: (i, 0)),
      out_shape=jax.ShapeDtypeStruct((_N, _U), jnp.float32),
  )(aggp, wsum3, W, b2)


@jax.jit
def _impl(node_features, edge_index, edge_weights, W, b):
  ei = edge_index.astype(jnp.int32)
  src4 = ei[1].reshape(_NW, _NSEG, _CPS, _C)
  dst4 = ei[0].reshape(_NW, _NSEG, _CPS, _C)
  ew4 = edge_weights.reshape(_NW, _NSEG, _CPS, _C)
  aggp, wsump = _sc_agg(node_features, src4, dst4, ew4)
  return _tc_finish(aggp, wsump.reshape(_NC, _N, 1), W, b.reshape(1, _U))


def kernel(node_features, edge_index, edge_weights, W, b):
  return _impl(node_features, edge_index, edge_weights, W, b)
